# baseline (speedup 1.0000x reference)
import jax
import jax.numpy as jnp
from jax.experimental import pallas as pl
from jax.experimental.pallas import tpu as pltpu

H = 32
W = 32
C = 256
B = 16
P = H * W
GRID = 50


def _tc_body(w_ref, o_ref, t_scratch, sems):
    t_scratch[...] = jnp.transpose(w_ref[...].reshape(P, C), (1, 0))
    copies = [
        pltpu.make_async_copy(t_scratch, o_ref.at[b], sems.at[b])
        for b in range(B)
    ]
    for cp in copies:
        cp.start()
    for cp in copies:
        cp.wait()


def kernel(mask, pos_embed_weight):
    bsz, h, w = mask.shape
    w3 = pos_embed_weight.reshape(GRID, GRID, C)
    out = pl.pallas_call(
        _tc_body,
        grid=(1,),
        in_specs=[pl.BlockSpec((H, W, C), lambda b: (0, 0, 0))],
        out_specs=pl.BlockSpec(memory_space=pl.ANY),
        out_shape=jax.ShapeDtypeStruct((B, C, P), jnp.float32),
        scratch_shapes=[
            pltpu.VMEM((C, P), jnp.float32),
            pltpu.SemaphoreType.DMA((B,)),
        ],
    )(w3)
    return out.reshape(bsz, C, h, w)
